# baseline (speedup 1.0000x reference)
import functools
import jax
import jax.numpy as jnp
from jax import lax
from jax.experimental import pallas as pl
from jax.experimental.pallas import tpu as pltpu
from jax.experimental.pallas import tpu_sc as plsc

PAD_ID = 0
EOS_ID = 2
BATCH = 4096
SEQ = 20
DIM = 64

NUM_CORES = 2
NUM_SUBCORES = 16
NW = NUM_CORES * NUM_SUBCORES
TOTAL = BATCH * SEQ
ROWS_PER_W = TOTAL // NW
LANES = 16
CHUNK = 256
VPC = CHUNK // LANES
NCHUNK = ROWS_PER_W // CHUNK
NBUF = 2
NITER = NCHUNK // NBUF

_mesh = plsc.VectorSubcoreMesh(
    core_axis_name="c", subcore_axis_name="s",
    num_cores=NUM_CORES, num_subcores=NUM_SUBCORES)


@functools.partial(
    pl.kernel,
    mesh=_mesh,
    out_type=jax.ShapeDtypeStruct((TOTAL, DIM), jnp.float32),
    scratch_types=[
        pltpu.VMEM((NBUF, CHUNK, DIM), jnp.float32),
        pltpu.SemaphoreType.DMA((NBUF,)),
        pltpu.SemaphoreType.DMA((NBUF,)),
    ],
    compiler_params=pltpu.CompilerParams(use_tc_tiling_on_sc=False),
)
def _sc_probe(table_hbm, out_hbm, rows_v, gsem, osem):
    wid = lax.axis_index("s") * NUM_CORES + lax.axis_index("c")
    base = wid * ROWS_PER_W

    def start_gathers(c, b):
        for k in range(VPC):
            vec = base + c * CHUNK + k * LANES + lax.iota(jnp.int32, 16)
            pltpu.async_copy(
                table_hbm.at[vec],
                rows_v.at[b, pl.ds(k * LANES, LANES)],
                gsem.at[b],
            )

    def drain_gathers(b):
        for k in range(VPC):
            pltpu.make_async_copy(
                table_hbm.at[lax.iota(jnp.int32, 16)],
                rows_v.at[b, pl.ds(0, LANES)],
                gsem.at[b],
            ).wait()

    def start_out(c, b):
        pltpu.async_copy(
            rows_v.at[b], out_hbm.at[pl.ds(base + c * CHUNK, CHUNK)],
            osem.at[b])

    def out_done(b):
        pltpu.make_async_copy(
            rows_v.at[b], out_hbm.at[pl.ds(base, CHUNK)], osem.at[b]).wait()

    def loop_body(t, carry):
        c0 = t * NBUF

        @pl.when(t > 0)
        def _():
            for b in range(NBUF):
                out_done(b)

        for b in range(NBUF):
            start_gathers(c0 + b, b)
        for b in range(NBUF):
            drain_gathers(b)
            start_out(c0 + b, b)
        return carry

    lax.fori_loop(0, NITER, loop_body, 0)
    for b in range(NBUF):
        out_done(b)


def kernel(lookup_ids, embedding_matrix):
    mask = lookup_ids == PAD_ID
    eos_positions = (lookup_ids == EOS_ID).astype(jnp.float32)
    flat_ids = lookup_ids.reshape(-1)
    matrices = jnp.take(embedding_matrix, flat_ids, axis=0)
    matrices = matrices.reshape(BATCH, SEQ, DIM)
    probe = _sc_probe(embedding_matrix)
    matrices = matrices.at[0, 0, 0].add(probe[0, 0] * 0.0)
    return (matrices, mask, eos_positions)


# speedup vs baseline: 1.0412x; 1.0412x over previous
import functools
import jax
import jax.numpy as jnp
from jax import lax
from jax.experimental import pallas as pl
from jax.experimental.pallas import tpu as pltpu
from jax.experimental.pallas import tpu_sc as plsc

PAD_ID = 0
EOS_ID = 2
BATCH = 4096
SEQ = 20
DIM = 64

NUM_CORES = 2
NUM_SUBCORES = 16
NW = NUM_CORES * NUM_SUBCORES
TOTAL = BATCH * SEQ
ROWS_PER_W = TOTAL // NW
LANES = 16
CHUNK = 128
VPC = CHUNK // LANES
NCHUNK = ROWS_PER_W // CHUNK
NBUF = 2
NITER = NCHUNK // NBUF

SAMPLES_PER_BLK = 128


_mesh = plsc.VectorSubcoreMesh(
    core_axis_name="c", subcore_axis_name="s",
    num_cores=NUM_CORES, num_subcores=NUM_SUBCORES)


@functools.partial(
    pl.kernel,
    mesh=_mesh,
    out_type=jax.ShapeDtypeStruct((TOTAL, 2 * DIM), jnp.float32),
    scratch_types=[
        pltpu.VMEM((ROWS_PER_W,), jnp.int32),
        pltpu.VMEM((NBUF, CHUNK, 2 * DIM), jnp.float32),
        pltpu.SemaphoreType.DMA((NBUF,)),
        pltpu.SemaphoreType.DMA((NBUF,)),
    ],
    compiler_params=pltpu.CompilerParams(use_tc_tiling_on_sc=True),
)
def _sc_gather(pidx_hbm, table_hbm, out_hbm, idx_v, rows_v, gsem, osem):
    wid = lax.axis_index("s") * NUM_CORES + lax.axis_index("c")
    base = wid * ROWS_PER_W
    pltpu.sync_copy(pidx_hbm.at[pl.ds(base, ROWS_PER_W)], idx_v)

    def start_gathers(c, b):
        for k in range(VPC):
            vec = idx_v[pl.ds(c * CHUNK + k * LANES, LANES)]
            pltpu.async_copy(
                table_hbm.at[vec],
                rows_v.at[b, pl.ds(k * LANES, LANES)],
                gsem.at[b],
            )

    def drain_gathers(b):
        for k in range(VPC):
            pltpu.make_async_copy(
                table_hbm.at[idx_v[pl.ds(0, LANES)]],
                rows_v.at[b, pl.ds(0, LANES)],
                gsem.at[b],
            ).wait()

    def start_out(c, b):
        pltpu.async_copy(
            rows_v.at[b], out_hbm.at[pl.ds(base + c * CHUNK, CHUNK)],
            osem.at[b])

    def out_done(b):
        pltpu.make_async_copy(
            rows_v.at[b], out_hbm.at[pl.ds(base, CHUNK)], osem.at[b]).wait()

    def loop_body(t, carry):
        c0 = t * NBUF

        @pl.when(t > 0)
        def _():
            for b in range(NBUF):
                out_done(b)

        for b in range(NBUF):
            start_gathers(c0 + b, b)
        for b in range(NBUF):
            drain_gathers(b)
            start_out(c0 + b, b)
        return carry

    lax.fori_loop(0, NITER, loop_body, 0)
    for b in range(NBUF):
        out_done(b)


def _finish_body(pairs_ref, ids_ref, mat_ref, mask_ref, eos_ref):
    ids = ids_ref[...]
    par = ids & 1
    cols = []
    for s in range(SEQ):
        xs = pairs_ref[s]
        sel = jnp.where(par[:, s:s + 1] == 1, xs[:, DIM:], xs[:, :DIM])
        cols.append(sel[:, None, :])
    mat_ref[...] = jnp.concatenate(cols, axis=1)
    mask_ref[...] = ids == PAD_ID
    eos_ref[...] = (ids == EOS_ID).astype(jnp.float32)


GB = 128

_finish_call = pl.pallas_call(
    _finish_body,
    grid=(BATCH // GB,),
    in_specs=[
        pl.BlockSpec((SEQ, GB, 2 * DIM), lambda j: (0, j, 0)),
        pl.BlockSpec((GB, SEQ), lambda j: (j, 0)),
    ],
    out_specs=(
        pl.BlockSpec((GB, SEQ, DIM), lambda j: (j, 0, 0)),
        pl.BlockSpec((GB, SEQ), lambda j: (j, 0)),
        pl.BlockSpec((GB, SEQ), lambda j: (j, 0)),
    ),
    out_shape=(
        jax.ShapeDtypeStruct((BATCH, SEQ, DIM), jnp.float32),
        jax.ShapeDtypeStruct((BATCH, SEQ), jnp.bool_),
        jax.ShapeDtypeStruct((BATCH, SEQ), jnp.float32),
    ),
)


def kernel(lookup_ids, embedding_matrix):
    flat_t = jnp.transpose(lookup_ids).reshape(-1)
    pidx = flat_t >> 1
    table2 = embedding_matrix.reshape(500000, 2 * DIM)
    pairs = _sc_gather(pidx, table2)
    pairs3 = pairs.reshape(SEQ, BATCH, 2 * DIM)
    matrices, mask, eos = _finish_call(pairs3, lookup_ids)
    return (matrices, mask, eos)


# speedup vs baseline: 1.0417x; 1.0005x over previous
import functools
import jax
import jax.numpy as jnp
from jax import lax
from jax.experimental import pallas as pl
from jax.experimental.pallas import tpu as pltpu
from jax.experimental.pallas import tpu_sc as plsc

PAD_ID = 0
EOS_ID = 2
BATCH = 4096
SEQ = 20
DIM = 64

NUM_CORES = 2
NUM_SUBCORES = 16
NW = NUM_CORES * NUM_SUBCORES
TOTAL = BATCH * SEQ
ROWS_PER_W = TOTAL // NW
LANES = 16
CHUNK = 256
VPC = CHUNK // LANES
NCHUNK = ROWS_PER_W // CHUNK
NBUF = 2
NITER = NCHUNK // NBUF


_mesh = plsc.VectorSubcoreMesh(
    core_axis_name="c", subcore_axis_name="s",
    num_cores=NUM_CORES, num_subcores=NUM_SUBCORES)


@functools.partial(
    pl.kernel,
    mesh=_mesh,
    out_type=jax.ShapeDtypeStruct((TOTAL, DIM), jnp.float32),
    scratch_types=[
        pltpu.VMEM((ROWS_PER_W,), jnp.int32),
        pltpu.VMEM((NBUF, CHUNK, DIM), jnp.float32),
        pltpu.SemaphoreType.DMA((NBUF,)),
        pltpu.SemaphoreType.DMA((NBUF,)),
    ],
    compiler_params=pltpu.CompilerParams(use_tc_tiling_on_sc=False),
)
def _sc_gather(ids_hbm, table_hbm, out_hbm, idx_v, rows_v, gsem, osem):
    wid = lax.axis_index("s") * NUM_CORES + lax.axis_index("c")
    base = wid * ROWS_PER_W
    pltpu.sync_copy(ids_hbm.at[pl.ds(base, ROWS_PER_W)], idx_v)

    def start_gathers(c, b):
        for k in range(VPC):
            vec = idx_v[pl.ds(c * CHUNK + k * LANES, LANES)]
            pltpu.async_copy(
                table_hbm.at[vec],
                rows_v.at[b, pl.ds(k * LANES, LANES)],
                gsem.at[b],
            )

    def drain_gathers(b):
        for k in range(VPC):
            pltpu.make_async_copy(
                table_hbm.at[idx_v[pl.ds(0, LANES)]],
                rows_v.at[b, pl.ds(0, LANES)],
                gsem.at[b],
            ).wait()

    def start_out(c, b):
        pltpu.async_copy(
            rows_v.at[b], out_hbm.at[pl.ds(base + c * CHUNK, CHUNK)],
            osem.at[b])

    def out_done(b):
        pltpu.make_async_copy(
            rows_v.at[b], out_hbm.at[pl.ds(base, CHUNK)], osem.at[b]).wait()

    def loop_body(t, carry):
        c0 = t * NBUF

        @pl.when(t > 0)
        def _():
            for b in range(NBUF):
                out_done(b)

        for b in range(NBUF):
            start_gathers(c0 + b, b)
        for b in range(NBUF):
            drain_gathers(b)
            start_out(c0 + b, b)
        return carry

    lax.fori_loop(0, NITER, loop_body, 0)
    for b in range(NBUF):
        out_done(b)


GB = 256


def _finish_body(rows_ref, ids_ref, mat_ref, mask_ref, eos_ref):
    ids = ids_ref[...]
    cols = []
    for s in range(SEQ):
        cols.append(rows_ref[s][:, None, :])
    mat_ref[...] = jnp.concatenate(cols, axis=1)
    mask_ref[...] = ids == PAD_ID
    eos_ref[...] = (ids == EOS_ID).astype(jnp.float32)


_finish_call = pl.pallas_call(
    _finish_body,
    grid=(BATCH // GB,),
    in_specs=[
        pl.BlockSpec((SEQ, GB, DIM), lambda j: (0, j, 0)),
        pl.BlockSpec((GB, SEQ), lambda j: (j, 0)),
    ],
    out_specs=(
        pl.BlockSpec((GB, SEQ, DIM), lambda j: (j, 0, 0)),
        pl.BlockSpec((GB, SEQ), lambda j: (j, 0)),
        pl.BlockSpec((GB, SEQ), lambda j: (j, 0)),
    ),
    out_shape=(
        jax.ShapeDtypeStruct((BATCH, SEQ, DIM), jnp.float32),
        jax.ShapeDtypeStruct((BATCH, SEQ), jnp.bool_),
        jax.ShapeDtypeStruct((BATCH, SEQ), jnp.float32),
    ),
)


def kernel(lookup_ids, embedding_matrix):
    flat_t = jnp.transpose(lookup_ids).reshape(-1)
    gathered = _sc_gather(flat_t, embedding_matrix)
    rows3 = gathered.reshape(SEQ, BATCH, DIM)
    matrices, mask, eos = _finish_call(rows3, lookup_ids)
    return (matrices, mask, eos)
